# Initial kernel scaffold; baseline (speedup 1.0000x reference)
import jax
import jax.numpy as jnp
from jax.experimental import pallas as pl


def kernel(x, edge_index, edge_label_index, W0, b0, W1, b1, W2, b2, g0, be0, g1, be1, g2, be2, Wp1, bp1, gp1, bep1, Wp2, bp2, gp2, bep2, Wp3, bp3, Wp4, bp4):
    raise NotImplementedError("write your pallas kernel here")


# speedup vs baseline: 6.1264x; 6.1264x over previous
import functools
import math

import jax
import jax.numpy as jnp
from jax import lax
from jax.experimental import pallas as pl
from jax.experimental.pallas import tpu as pltpu
from jax.experimental.pallas import tpu_sc as plsc

EPS = 1e-5
BNK = 1.0 / math.sqrt(1.0 + EPS)

NC = 2
NS = 16
NW = NC * NS
C = 128

@functools.lru_cache(maxsize=None)
def _mesh():
    return plsc.VectorSubcoreMesh(core_axis_name="c", subcore_axis_name="s",
                                  num_cores=NC, num_subcores=NS)


@functools.lru_cache(maxsize=None)
def _deg_kernel(acc_n, ep, w):
    chunks = ep // (NW * C)
    rpt = acc_n // NS

    def body(col_hbm, ones_hbm, zero_hbm, out_hbm, acc, idx_v, ones_v, sem):
        c = lax.axis_index("c")
        s = lax.axis_index("s")
        wid = s * NC + c
        pltpu.sync_copy(zero_hbm.at[pl.ds(s * rpt, rpt)],
                        acc.at[pl.ds(s * rpt, rpt)])
        pltpu.sync_copy(ones_hbm, ones_v)
        plsc.subcore_barrier()

        def chunk(i, carry):
            base = (wid * chunks + i) * C
            pltpu.sync_copy(col_hbm.at[pl.ds(base, C)], idx_v)
            pltpu.sync_copy(ones_v, acc.at[idx_v], add=True)
            return carry

        lax.fori_loop(0, chunks, chunk, 0)
        plsc.subcore_barrier()
        pltpu.sync_copy(acc.at[pl.ds(s * rpt, rpt)],
                        out_hbm.at[c].at[pl.ds(s * rpt, rpt)])

    return pl.kernel(
        body,
        out_type=jax.ShapeDtypeStruct((NC, acc_n, w), jnp.float32),
        mesh=_mesh(),
        scratch_types=[
            pltpu.VMEM_SHARED((acc_n, w), jnp.float32),
            pltpu.VMEM((C,), jnp.int32),
            pltpu.VMEM((C, w), jnp.float32),
            pltpu.SemaphoreType.DMA,
        ],
    )


@functools.lru_cache(maxsize=None)
def _prop_kernel(acc_n, ep, h):
    chunks = ep // (NW * C)
    rpt = acc_n // NS

    def body(u_hbm, row_hbm, col_hbm, zero_hbm, out_hbm,
             acc, ridx, cidx, buf, sem):
        c = lax.axis_index("c")
        s = lax.axis_index("s")
        wid = s * NC + c
        pltpu.sync_copy(zero_hbm.at[pl.ds(s * rpt, rpt)],
                        acc.at[pl.ds(s * rpt, rpt)])
        plsc.subcore_barrier()

        def chunk(i, carry):
            base = (wid * chunks + i) * C
            pltpu.sync_copy(row_hbm.at[pl.ds(base, C)], ridx)
            pltpu.sync_copy(col_hbm.at[pl.ds(base, C)], cidx)
            pltpu.async_copy(u_hbm.at[ridx], buf, sem).wait()
            pltpu.sync_copy(buf, acc.at[cidx], add=True)
            return carry

        lax.fori_loop(0, chunks, chunk, 0)
        plsc.subcore_barrier()
        pltpu.sync_copy(acc.at[pl.ds(s * rpt, rpt)],
                        out_hbm.at[c].at[pl.ds(s * rpt, rpt)])

    return pl.kernel(
        body,
        out_type=jax.ShapeDtypeStruct((NC, acc_n, h), jnp.float32),
        mesh=_mesh(),
        scratch_types=[
            pltpu.VMEM_SHARED((acc_n, h), jnp.float32),
            pltpu.VMEM((C,), jnp.int32),
            pltpu.VMEM((C,), jnp.int32),
            pltpu.VMEM((C, h), jnp.float32),
            pltpu.SemaphoreType.DMA,
        ],
    )


@functools.lru_cache(maxsize=None)
def _pair_gather_kernel(elp, n, h):
    chunks = elp // (NW * C)

    def body(z_hbm, src_hbm, dst_hbm, zs_hbm, zd_hbm, idx_v, buf, sem):
        c = lax.axis_index("c")
        s = lax.axis_index("s")
        wid = s * NC + c

        def chunk(i, carry):
            base = (wid * chunks + i) * C
            pltpu.sync_copy(src_hbm.at[pl.ds(base, C)], idx_v)
            pltpu.async_copy(z_hbm.at[idx_v], buf, sem).wait()
            pltpu.sync_copy(buf, zs_hbm.at[pl.ds(base, C)])
            pltpu.sync_copy(dst_hbm.at[pl.ds(base, C)], idx_v)
            pltpu.async_copy(z_hbm.at[idx_v], buf, sem).wait()
            pltpu.sync_copy(buf, zd_hbm.at[pl.ds(base, C)])
            return carry

        lax.fori_loop(0, chunks, chunk, 0)

    return pl.kernel(
        body,
        out_type=(jax.ShapeDtypeStruct((elp, h), jnp.float32),
                  jax.ShapeDtypeStruct((elp, h), jnp.float32)),
        mesh=_mesh(),
        scratch_types=[
            pltpu.VMEM((C,), jnp.int32),
            pltpu.VMEM((C, h), jnp.float32),
            pltpu.SemaphoreType.DMA,
        ],
    )


def _row_block(n):
    for blk in (2000, 1000, 500, 200, 40, 8):
        if n % blk == 0:
            return blk
    return n


def _tc0_call(x, dega, degb, w0, nblk):
    n, d = x.shape
    h = w0.shape[1]
    grid = n // nblk

    def body(x_ref, da_ref, db_ref, w_ref, u_ref, dinv_ref):
        deg = da_ref[...] + db_ref[...] + 1.0
        dinv = lax.rsqrt(deg)
        hm = jnp.dot(x_ref[...], w_ref[...],
                     preferred_element_type=jnp.float32)
        u_ref[...] = hm * dinv
        dinv_ref[...] = dinv

    return pl.pallas_call(
        body,
        grid=(grid,),
        in_specs=[
            pl.BlockSpec((nblk, d), lambda i: (i, 0)),
            pl.BlockSpec((nblk, 1), lambda i: (i, 0)),
            pl.BlockSpec((nblk, 1), lambda i: (i, 0)),
            pl.BlockSpec((d, h), lambda i: (0, 0)),
        ],
        out_specs=[
            pl.BlockSpec((nblk, h), lambda i: (i, 0)),
            pl.BlockSpec((nblk, 1), lambda i: (i, 0)),
        ],
        out_shape=[
            jax.ShapeDtypeStruct((n, h), jnp.float32),
            jax.ShapeDtypeStruct((n, 1), jnp.float32),
        ],
    )(x, dega, degb, w0)


def _tc_layer_call(sa, sb, u, dinv, w, b, g, be, nblk, relu, matmul):
    n, h = u.shape
    grid = n // nblk
    ho = w.shape[1] if matmul else h

    def body(sa_ref, sb_ref, u_ref, dinv_ref, w_ref, b_ref, g_ref, be_ref,
             out_ref):
        dinv_v = dinv_ref[...]
        z = dinv_v * (sa_ref[...] + sb_ref[...] + u_ref[...]) + b_ref[...]
        z = z * (g_ref[...] * BNK) + be_ref[...]
        if relu:
            z = jnp.maximum(z, 0.0)
        if matmul:
            out_ref[...] = jnp.dot(z, w_ref[...],
                                   preferred_element_type=jnp.float32) * dinv_v
        else:
            out_ref[...] = z

    return pl.pallas_call(
        body,
        grid=(grid,),
        in_specs=[
            pl.BlockSpec((nblk, h), lambda i: (i, 0)),
            pl.BlockSpec((nblk, h), lambda i: (i, 0)),
            pl.BlockSpec((nblk, h), lambda i: (i, 0)),
            pl.BlockSpec((nblk, 1), lambda i: (i, 0)),
            pl.BlockSpec(w.shape, lambda i: (0, 0)),
            pl.BlockSpec((1, h), lambda i: (0, 0)),
            pl.BlockSpec((1, h), lambda i: (0, 0)),
            pl.BlockSpec((1, h), lambda i: (0, 0)),
        ],
        out_specs=pl.BlockSpec((nblk, ho), lambda i: (i, 0)),
        out_shape=jax.ShapeDtypeStruct((n, ho), jnp.float32),
    )(sa, sb, u, dinv, w, b, g, be)


def _mlp_call(zs, zd, w1a, w1b, b1, g1, e1, w2, b2, g2, e2, w3, b3, w4, b4,
              blk):
    m, h = zs.shape
    grid = m // blk
    h2 = w1a.shape[1]
    h3 = w2.shape[1]
    h4 = w3.shape[1]

    def body(zs_ref, zd_ref, w1a_ref, w1b_ref, b1_ref, g1_ref, e1_ref,
             w2_ref, b2_ref, g2_ref, e2_ref, w3_ref, b3_ref, w4_ref, b4_ref,
             out_ref):
        t = (jnp.dot(zs_ref[...], w1a_ref[...],
                     preferred_element_type=jnp.float32)
             + jnp.dot(zd_ref[...], w1b_ref[...],
                       preferred_element_type=jnp.float32) + b1_ref[...])
        t = t * (g1_ref[...] * BNK) + e1_ref[...]
        t = jnp.maximum(t, 0.0)
        t = jnp.dot(t, w2_ref[...],
                    preferred_element_type=jnp.float32) + b2_ref[...]
        t = t * (g2_ref[...] * BNK) + e2_ref[...]
        t = jnp.maximum(t, 0.0)
        t = jnp.dot(t, w3_ref[...],
                    preferred_element_type=jnp.float32) + b3_ref[...]
        t = jnp.maximum(t, 0.0)
        out_ref[...] = jnp.dot(t, w4_ref[...],
                               preferred_element_type=jnp.float32) + b4_ref[...]

    full = lambda a: pl.BlockSpec(a.shape, lambda i: (0, 0))
    return pl.pallas_call(
        body,
        grid=(grid,),
        in_specs=[
            pl.BlockSpec((blk, h), lambda i: (i, 0)),
            pl.BlockSpec((blk, h), lambda i: (i, 0)),
            full(w1a), full(w1b), full(b1), full(g1), full(e1),
            full(w2), full(b2), full(g2), full(e2),
            full(w3), full(b3), full(w4), full(b4),
        ],
        out_specs=pl.BlockSpec((blk, 1), lambda i: (i, 0)),
        out_shape=jax.ShapeDtypeStruct((m, 1), jnp.float32),
    )(zs, zd, w1a, w1b, b1, g1, e1, w2, b2, g2, e2, w3, b3, w4, b4)


def kernel(x, edge_index, edge_label_index, W0, b0, W1, b1, W2, b2,
           g0, be0, g1, be1, g2, be2,
           Wp1, bp1, gp1, bep1, Wp2, bp2, gp2, bep2, Wp3, bp3, Wp4, bp4):
    n, d = x.shape
    h = W0.shape[1]
    e = edge_index.shape[1]
    el = edge_label_index.shape[1]

    row = edge_index[0].astype(jnp.int32)
    col = edge_index[1].astype(jnp.int32)
    ep = -(-e // (NW * C)) * (NW * C)
    rowp = jnp.concatenate([row, jnp.zeros((ep - e,), jnp.int32)])
    colp = jnp.concatenate([col, jnp.full((ep - e,), n, jnp.int32)])

    acc_n = -(-(n + 1) // 128) * 128

    ones = jnp.ones((C, h), jnp.float32)
    zeroh = jnp.zeros((acc_n, h), jnp.float32)

    dego = _deg_kernel(acc_n, ep, h)(colp, ones, zeroh)
    dega = dego[0, :n, 0:1]
    degb = dego[1, :n, 0:1]

    nblk = _row_block(n)
    u, dinv = _tc0_call(x, dega, degb, W0, nblk)

    prop = _prop_kernel(acc_n, ep, h)
    layer_params = [(W1, b0, g0, be0), (W2, b1, g1, be1)]
    for w, b, g, be in layer_params:
        po = prop(u, rowp, colp, zeroh)
        u = _tc_layer_call(po[0, :n], po[1, :n], u, dinv, w,
                           b.reshape(1, -1), g.reshape(1, -1),
                           be.reshape(1, -1), nblk, relu=True, matmul=True)
    po = prop(u, rowp, colp, zeroh)
    z = _tc_layer_call(po[0, :n], po[1, :n], u, dinv, W2,
                       b2.reshape(1, -1), g2.reshape(1, -1),
                       be2.reshape(1, -1), nblk, relu=False, matmul=False)

    src = edge_label_index[0].astype(jnp.int32)
    dst = edge_label_index[1].astype(jnp.int32)
    elp = -(-el // (NW * C)) * (NW * C)
    srcp = jnp.concatenate([src, jnp.zeros((elp - el,), jnp.int32)])
    dstp = jnp.concatenate([dst, jnp.zeros((elp - el,), jnp.int32)])

    zs, zd = _pair_gather_kernel(elp, n, h)(z, srcp, dstp)

    blk = 512 if elp % 512 == 0 else C
    out = _mlp_call(
        zs, zd, Wp1[:h], Wp1[h:], bp1.reshape(1, -1), gp1.reshape(1, -1),
        bep1.reshape(1, -1), Wp2, bp2.reshape(1, -1), gp2.reshape(1, -1),
        bep2.reshape(1, -1), Wp3, bp3.reshape(1, -1), Wp4,
        bp4.reshape(1, -1), blk)
    return out.reshape(-1)[:el]


# speedup vs baseline: 7.4700x; 1.2193x over previous
import functools
import math

import jax
import jax.numpy as jnp
from jax import lax
from jax.experimental import pallas as pl
from jax.experimental.pallas import tpu as pltpu
from jax.experimental.pallas import tpu_sc as plsc

EPS = 1e-5
BNK = 1.0 / math.sqrt(1.0 + EPS)

NC = 2
NS = 16
NW = NC * NS
C = 128

@functools.lru_cache(maxsize=None)
def _mesh():
    return plsc.VectorSubcoreMesh(core_axis_name="c", subcore_axis_name="s",
                                  num_cores=NC, num_subcores=NS)


@functools.lru_cache(maxsize=None)
def _deg_kernel(acc_n, ep, w):
    chunks = ep // (NW * C)
    rpt = acc_n // NS

    def body(col_hbm, ones_hbm, zero_hbm, out_hbm, acc, idx_v, ones_v, sem):
        c = lax.axis_index("c")
        s = lax.axis_index("s")
        wid = s * NC + c
        pltpu.sync_copy(zero_hbm.at[pl.ds(s * rpt, rpt)],
                        acc.at[pl.ds(s * rpt, rpt)])
        pltpu.sync_copy(ones_hbm, ones_v)
        plsc.subcore_barrier()

        def chunk(i, carry):
            base = (wid * chunks + i) * C
            pltpu.sync_copy(col_hbm.at[pl.ds(base, C)], idx_v)
            pltpu.sync_copy(ones_v, acc.at[idx_v], add=True)
            return carry

        lax.fori_loop(0, chunks, chunk, 0)
        plsc.subcore_barrier()
        pltpu.sync_copy(acc.at[pl.ds(s * rpt, rpt)],
                        out_hbm.at[c].at[pl.ds(s * rpt, rpt)])

    return pl.kernel(
        body,
        out_type=jax.ShapeDtypeStruct((NC, acc_n, w), jnp.float32),
        mesh=_mesh(),
        scratch_types=[
            pltpu.VMEM_SHARED((acc_n, w), jnp.float32),
            pltpu.VMEM((C,), jnp.int32),
            pltpu.VMEM((C, w), jnp.float32),
            pltpu.SemaphoreType.DMA,
        ],
    )


@functools.lru_cache(maxsize=None)
def _prop_kernel(acc_n, ep, h):
    chunks = ep // (NW * C)
    rpt = acc_n // NS

    half = chunks // 2

    def body(u_hbm, row_hbm, col_hbm, zero_hbm, out_hbm,
             acc, ridx0, cidx0, ridx1, cidx1, buf0, buf1, sem0, sem1):
        c = lax.axis_index("c")
        s = lax.axis_index("s")
        wid = s * NC + c
        base0 = wid * chunks * C
        pltpu.sync_copy(zero_hbm.at[pl.ds(s * rpt, rpt)],
                        acc.at[pl.ds(s * rpt, rpt)])
        plsc.subcore_barrier()

        pltpu.sync_copy(row_hbm.at[pl.ds(base0, C)], ridx0)
        pltpu.sync_copy(col_hbm.at[pl.ds(base0, C)], cidx0)
        pltpu.async_copy(u_hbm.at[ridx0], buf0, sem0)

        def pair(j, carry):
            b1 = base0 + (2 * j + 1) * C
            pltpu.sync_copy(row_hbm.at[pl.ds(b1, C)], ridx1)
            pltpu.sync_copy(col_hbm.at[pl.ds(b1, C)], cidx1)
            pltpu.async_copy(u_hbm.at[ridx1], buf1, sem1)
            pltpu.make_async_copy(u_hbm.at[ridx0], buf0, sem0).wait()
            pltpu.sync_copy(buf0, acc.at[cidx0], add=True)

            @pl.when(j + 1 < half)
            def _():
                b2 = base0 + (2 * j + 2) * C
                pltpu.sync_copy(row_hbm.at[pl.ds(b2, C)], ridx0)
                pltpu.sync_copy(col_hbm.at[pl.ds(b2, C)], cidx0)
                pltpu.async_copy(u_hbm.at[ridx0], buf0, sem0)

            pltpu.make_async_copy(u_hbm.at[ridx1], buf1, sem1).wait()
            pltpu.sync_copy(buf1, acc.at[cidx1], add=True)
            return carry

        lax.fori_loop(0, half, pair, 0)
        if chunks % 2:
            b = base0 + (chunks - 1) * C
            pltpu.sync_copy(row_hbm.at[pl.ds(b, C)], ridx1)
            pltpu.sync_copy(col_hbm.at[pl.ds(b, C)], cidx1)
            pltpu.async_copy(u_hbm.at[ridx1], buf1, sem1).wait()
            pltpu.sync_copy(buf1, acc.at[cidx1], add=True)
        plsc.subcore_barrier()
        pltpu.sync_copy(acc.at[pl.ds(s * rpt, rpt)],
                        out_hbm.at[c].at[pl.ds(s * rpt, rpt)])

    return pl.kernel(
        body,
        out_type=jax.ShapeDtypeStruct((NC, acc_n, h), jnp.float32),
        mesh=_mesh(),
        scratch_types=[
            pltpu.VMEM_SHARED((acc_n, h), jnp.float32),
            pltpu.VMEM((C,), jnp.int32),
            pltpu.VMEM((C,), jnp.int32),
            pltpu.VMEM((C,), jnp.int32),
            pltpu.VMEM((C,), jnp.int32),
            pltpu.VMEM((C, h), jnp.float32),
            pltpu.VMEM((C, h), jnp.float32),
            pltpu.SemaphoreType.DMA,
            pltpu.SemaphoreType.DMA,
        ],
    )


@functools.lru_cache(maxsize=None)
def _pair_gather_kernel(elp, n, h):
    chunks = elp // (NW * C)

    def body(z_hbm, src_hbm, dst_hbm, zs_hbm, zd_hbm,
             sidx, didx, buf0, buf1, sem0, sem1):
        c = lax.axis_index("c")
        s = lax.axis_index("s")
        wid = s * NC + c
        base0 = wid * chunks * C

        pltpu.sync_copy(src_hbm.at[pl.ds(base0, C)], sidx)
        pltpu.async_copy(z_hbm.at[sidx], buf0, sem0)

        def chunk(j, carry):
            base = base0 + j * C
            pltpu.sync_copy(dst_hbm.at[pl.ds(base, C)], didx)
            pltpu.async_copy(z_hbm.at[didx], buf1, sem1)
            pltpu.make_async_copy(z_hbm.at[sidx], buf0, sem0).wait()
            pltpu.sync_copy(buf0, zs_hbm.at[pl.ds(base, C)])

            @pl.when(j + 1 < chunks)
            def _():
                nbase = base + C
                pltpu.sync_copy(src_hbm.at[pl.ds(nbase, C)], sidx)
                pltpu.async_copy(z_hbm.at[sidx], buf0, sem0)

            pltpu.make_async_copy(z_hbm.at[didx], buf1, sem1).wait()
            pltpu.sync_copy(buf1, zd_hbm.at[pl.ds(base, C)])
            return carry

        lax.fori_loop(0, chunks, chunk, 0)

    return pl.kernel(
        body,
        out_type=(jax.ShapeDtypeStruct((elp, h), jnp.float32),
                  jax.ShapeDtypeStruct((elp, h), jnp.float32)),
        mesh=_mesh(),
        scratch_types=[
            pltpu.VMEM((C,), jnp.int32),
            pltpu.VMEM((C,), jnp.int32),
            pltpu.VMEM((C, h), jnp.float32),
            pltpu.VMEM((C, h), jnp.float32),
            pltpu.SemaphoreType.DMA,
            pltpu.SemaphoreType.DMA,
        ],
    )


def _row_block(n):
    for blk in (2000, 1000, 500, 200, 40, 8):
        if n % blk == 0:
            return blk
    return n


def _tc0_call(x, dega, degb, w0, nblk):
    n, d = x.shape
    h = w0.shape[1]
    grid = n // nblk

    def body(x_ref, da_ref, db_ref, w_ref, u_ref, dinv_ref):
        deg = da_ref[...] + db_ref[...] + 1.0
        dinv = lax.rsqrt(deg)
        hm = jnp.dot(x_ref[...], w_ref[...],
                     preferred_element_type=jnp.float32)
        u_ref[...] = hm * dinv
        dinv_ref[...] = dinv

    return pl.pallas_call(
        body,
        grid=(grid,),
        in_specs=[
            pl.BlockSpec((nblk, d), lambda i: (i, 0)),
            pl.BlockSpec((nblk, 1), lambda i: (i, 0)),
            pl.BlockSpec((nblk, 1), lambda i: (i, 0)),
            pl.BlockSpec((d, h), lambda i: (0, 0)),
        ],
        out_specs=[
            pl.BlockSpec((nblk, h), lambda i: (i, 0)),
            pl.BlockSpec((nblk, 1), lambda i: (i, 0)),
        ],
        out_shape=[
            jax.ShapeDtypeStruct((n, h), jnp.float32),
            jax.ShapeDtypeStruct((n, 1), jnp.float32),
        ],
    )(x, dega, degb, w0)


def _tc_layer_call(sa, sb, u, dinv, w, b, g, be, nblk, relu, matmul):
    n, h = u.shape
    grid = n // nblk
    ho = w.shape[1] if matmul else h

    def body(sa_ref, sb_ref, u_ref, dinv_ref, w_ref, b_ref, g_ref, be_ref,
             out_ref):
        dinv_v = dinv_ref[...]
        z = dinv_v * (sa_ref[...] + sb_ref[...] + u_ref[...]) + b_ref[...]
        z = z * (g_ref[...] * BNK) + be_ref[...]
        if relu:
            z = jnp.maximum(z, 0.0)
        if matmul:
            out_ref[...] = jnp.dot(z, w_ref[...],
                                   preferred_element_type=jnp.float32) * dinv_v
        else:
            out_ref[...] = z

    return pl.pallas_call(
        body,
        grid=(grid,),
        in_specs=[
            pl.BlockSpec((nblk, h), lambda i: (i, 0)),
            pl.BlockSpec((nblk, h), lambda i: (i, 0)),
            pl.BlockSpec((nblk, h), lambda i: (i, 0)),
            pl.BlockSpec((nblk, 1), lambda i: (i, 0)),
            pl.BlockSpec(w.shape, lambda i: (0, 0)),
            pl.BlockSpec((1, h), lambda i: (0, 0)),
            pl.BlockSpec((1, h), lambda i: (0, 0)),
            pl.BlockSpec((1, h), lambda i: (0, 0)),
        ],
        out_specs=pl.BlockSpec((nblk, ho), lambda i: (i, 0)),
        out_shape=jax.ShapeDtypeStruct((n, ho), jnp.float32),
    )(sa, sb, u, dinv, w, b, g, be)


def _mlp_call(zs, zd, w1a, w1b, b1, g1, e1, w2, b2, g2, e2, w3, b3, w4, b4,
              blk):
    m, h = zs.shape
    grid = m // blk
    h2 = w1a.shape[1]
    h3 = w2.shape[1]
    h4 = w3.shape[1]

    def body(zs_ref, zd_ref, w1a_ref, w1b_ref, b1_ref, g1_ref, e1_ref,
             w2_ref, b2_ref, g2_ref, e2_ref, w3_ref, b3_ref, w4_ref, b4_ref,
             out_ref):
        t = (jnp.dot(zs_ref[...], w1a_ref[...],
                     preferred_element_type=jnp.float32)
             + jnp.dot(zd_ref[...], w1b_ref[...],
                       preferred_element_type=jnp.float32) + b1_ref[...])
        t = t * (g1_ref[...] * BNK) + e1_ref[...]
        t = jnp.maximum(t, 0.0)
        t = jnp.dot(t, w2_ref[...],
                    preferred_element_type=jnp.float32) + b2_ref[...]
        t = t * (g2_ref[...] * BNK) + e2_ref[...]
        t = jnp.maximum(t, 0.0)
        t = jnp.dot(t, w3_ref[...],
                    preferred_element_type=jnp.float32) + b3_ref[...]
        t = jnp.maximum(t, 0.0)
        out_ref[...] = jnp.dot(t, w4_ref[...],
                               preferred_element_type=jnp.float32) + b4_ref[...]

    full = lambda a: pl.BlockSpec(a.shape, lambda i: (0, 0))
    return pl.pallas_call(
        body,
        grid=(grid,),
        in_specs=[
            pl.BlockSpec((blk, h), lambda i: (i, 0)),
            pl.BlockSpec((blk, h), lambda i: (i, 0)),
            full(w1a), full(w1b), full(b1), full(g1), full(e1),
            full(w2), full(b2), full(g2), full(e2),
            full(w3), full(b3), full(w4), full(b4),
        ],
        out_specs=pl.BlockSpec((blk, 1), lambda i: (i, 0)),
        out_shape=jax.ShapeDtypeStruct((m, 1), jnp.float32),
    )(zs, zd, w1a, w1b, b1, g1, e1, w2, b2, g2, e2, w3, b3, w4, b4)


def kernel(x, edge_index, edge_label_index, W0, b0, W1, b1, W2, b2,
           g0, be0, g1, be1, g2, be2,
           Wp1, bp1, gp1, bep1, Wp2, bp2, gp2, bep2, Wp3, bp3, Wp4, bp4):
    n, d = x.shape
    h = W0.shape[1]
    e = edge_index.shape[1]
    el = edge_label_index.shape[1]

    row = edge_index[0].astype(jnp.int32)
    col = edge_index[1].astype(jnp.int32)
    ep = -(-e // (NW * C)) * (NW * C)
    rowp = jnp.concatenate([row, jnp.zeros((ep - e,), jnp.int32)])
    colp = jnp.concatenate([col, jnp.full((ep - e,), n, jnp.int32)])

    acc_n = -(-(n + 1) // 128) * 128

    ones = jnp.ones((C, h), jnp.float32)
    zeroh = jnp.zeros((acc_n, h), jnp.float32)

    dego = _deg_kernel(acc_n, ep, h)(colp, ones, zeroh)
    dega = dego[0, :n, 0:1]
    degb = dego[1, :n, 0:1]

    nblk = _row_block(n)
    u, dinv = _tc0_call(x, dega, degb, W0, nblk)

    prop = _prop_kernel(acc_n, ep, h)
    layer_params = [(W1, b0, g0, be0), (W2, b1, g1, be1)]
    for w, b, g, be in layer_params:
        po = prop(u, rowp, colp, zeroh)
        u = _tc_layer_call(po[0, :n], po[1, :n], u, dinv, w,
                           b.reshape(1, -1), g.reshape(1, -1),
                           be.reshape(1, -1), nblk, relu=True, matmul=True)
    po = prop(u, rowp, colp, zeroh)
    z = _tc_layer_call(po[0, :n], po[1, :n], u, dinv, W2,
                       b2.reshape(1, -1), g2.reshape(1, -1),
                       be2.reshape(1, -1), nblk, relu=False, matmul=False)

    src = edge_label_index[0].astype(jnp.int32)
    dst = edge_label_index[1].astype(jnp.int32)
    elp = -(-el // (NW * C)) * (NW * C)
    srcp = jnp.concatenate([src, jnp.zeros((elp - el,), jnp.int32)])
    dstp = jnp.concatenate([dst, jnp.zeros((elp - el,), jnp.int32)])

    zs, zd = _pair_gather_kernel(elp, n, h)(z, srcp, dstp)

    blk = 512 if elp % 512 == 0 else C
    out = _mlp_call(
        zs, zd, Wp1[:h], Wp1[h:], bp1.reshape(1, -1), gp1.reshape(1, -1),
        bep1.reshape(1, -1), Wp2, bp2.reshape(1, -1), gp2.reshape(1, -1),
        bep2.reshape(1, -1), Wp3, bp3.reshape(1, -1), Wp4,
        bp4.reshape(1, -1), blk)
    return out.reshape(-1)[:el]
